# baseline (speedup 1.0000x reference)
import functools

import jax
import jax.numpy as jnp
from jax import lax
from jax.experimental import pallas as pl
from jax.experimental.pallas import tpu as pltpu
from jax.experimental.pallas import tpu_sc as plsc

N_NODES = 10000
N_EDGES = 320000
D_FEAT = 128
D_HID = 128
N_CLASSES = 64
ALPHA = 1.0

NW = 32
K = 128
C = 80
E_PAD = NW * C * K
N_PAD = 10112
R = N_PAD // 16
EPW = E_PAD // NW
MB = 1264

AK = 32
SUB = K // AK
AC = EPW // AK
NBUF = 4


def _sc_mesh():
    return plsc.VectorSubcoreMesh(
        core_axis_name="c", subcore_axis_name="s", num_cores=2, num_subcores=16
    )


def _deg_sc(dst3, ones128, zeros128):
    @functools.partial(
        pl.kernel,
        out_type=jax.ShapeDtypeStruct((2, N_PAD, 128), jnp.float32),
        mesh=_sc_mesh(),
        scratch_types=[
            pltpu.VMEM_SHARED((N_PAD, 128), jnp.float32),
            pltpu.VMEM((C, K), jnp.int32),
            pltpu.VMEM((K, 128), jnp.float32),
        ],
    )
    def body(dst_hbm, ones_hbm, zeros_hbm, out_hbm, dacc, dst_v, ones_v):
        cid = lax.axis_index("c")
        sid = lax.axis_index("s")
        wid = cid * 16 + sid
        pltpu.sync_copy(zeros_hbm.at[pl.ds(sid * R, R)], dacc.at[pl.ds(sid * R, R)])
        pltpu.sync_copy(dst_hbm.at[wid], dst_v)
        pltpu.sync_copy(ones_hbm, ones_v)
        plsc.subcore_barrier()

        def step(j, carry):
            pltpu.sync_copy(ones_v, dacc.at[dst_v.at[j]], add=True)
            return carry

        lax.fori_loop(0, C, step, 0)
        plsc.subcore_barrier()
        pltpu.sync_copy(
            dacc.at[pl.ds(sid * R, R)], out_hbm.at[cid, pl.ds(sid * R, R)]
        )

    return body(dst3, ones128, zeros128)


def _agg_sc(hs, src3, dst3, zeros128):
    @functools.partial(
        pl.kernel,
        out_type=jax.ShapeDtypeStruct((2, N_PAD, 128), jnp.float32),
        mesh=_sc_mesh(),
        scratch_types=[
            pltpu.VMEM_SHARED((N_PAD, 128), jnp.float32),
            pltpu.VMEM((C, K), jnp.int32),
            pltpu.VMEM((C, K), jnp.int32),
            pltpu.VMEM((NBUF, AK, 128), jnp.float32),
            [pltpu.SemaphoreType.DMA] * NBUF,
            [pltpu.SemaphoreType.DMA] * NBUF,
        ],
    )
    def body(hs_hbm, src_hbm, dst_hbm, zeros_hbm, out_hbm, acc, src_v, dst_v,
             rows_v, gsems, ssems):
        cid = lax.axis_index("c")
        sid = lax.axis_index("s")
        wid = cid * 16 + sid
        pltpu.sync_copy(zeros_hbm.at[pl.ds(sid * R, R)], acc.at[pl.ds(sid * R, R)])
        pltpu.sync_copy(src_hbm.at[wid], src_v)
        pltpu.sync_copy(dst_hbm.at[wid], dst_v)
        plsc.subcore_barrier()

        def idx_slice(ref, j):
            return ref.at[j // SUB, pl.ds((j % SUB) * AK, AK)]

        def start_gather(j, b):
            pltpu.async_copy(hs_hbm.at[idx_slice(src_v, j)], rows_v.at[b],
                             gsems[b])

        def wait_gather(j, b):
            pltpu.make_async_copy(hs_hbm.at[idx_slice(src_v, j)],
                                  rows_v.at[b], gsems[b]).wait()

        def start_scatter(j, b):
            pltpu.async_copy(rows_v.at[b], acc.at[idx_slice(dst_v, j)],
                             ssems[b], add=True)

        def wait_scatter(j, b):
            pltpu.make_async_copy(rows_v.at[b], acc.at[idx_slice(dst_v, j)],
                                  ssems[b]).wait()

        for b in range(NBUF):
            start_gather(b, b)

        def step(g, carry):
            for b in range(NBUF):
                j = g * NBUF + b
                wait_gather(j, b)
                start_scatter(j, b)
            for b in range(NBUF):
                j = g * NBUF + b
                wait_scatter(j, b)
                start_gather(j + NBUF, b)
            return carry

        lax.fori_loop(0, AC // NBUF - 1, step, 0)
        for b in range(NBUF):
            j = AC - NBUF + b
            wait_gather(j, b)
            start_scatter(j, b)
        for b in range(NBUF):
            j = AC - NBUF + b
            wait_scatter(j, b)

        plsc.subcore_barrier()
        pltpu.sync_copy(
            acc.at[pl.ds(sid * R, R)], out_hbm.at[cid, pl.ds(sid * R, R)]
        )

    return body(hs, src3, dst3, zeros128)


def _inv_from_deg(d0, d1):
    deg = jnp.maximum(d0[:, 0:1] + d1[:, 0:1], 1.0)
    return lax.rsqrt(deg)


def _matmul_t(x, w):
    return lax.dot_general(
        x, w, (((1,), (1,)), ((), ())), preferred_element_type=jnp.float32
    )


def _tc_h0(x, w1):
    def body(x_ref, w_ref, h0_ref):
        h0_ref[...] = jnp.maximum(_matmul_t(x_ref[...], w_ref[...]), 0.0)

    return pl.pallas_call(
        body,
        grid=(10,),
        in_specs=[
            pl.BlockSpec((1000, 128), lambda i: (i, 0)),
            pl.BlockSpec((128, 128), lambda i: (0, 0)),
        ],
        out_specs=pl.BlockSpec((1000, 128), lambda i: (i, 0)),
        out_shape=jax.ShapeDtypeStruct((N_NODES, 128), jnp.float32),
    )(x, w1)


def _tc_scale(h0, degp):
    def body(h0_ref, d0_ref, d1_ref, h0s_ref):
        inv = _inv_from_deg(d0_ref[0], d1_ref[0])
        h0s_ref[...] = h0_ref[...] * inv

    return pl.pallas_call(
        body,
        grid=(10,),
        in_specs=[
            pl.BlockSpec((1000, 128), lambda i: (i, 0)),
            pl.BlockSpec((1, 1000, 128), lambda i: (0, i, 0)),
            pl.BlockSpec((1, 1000, 128), lambda i: (1, i, 0)),
        ],
        out_specs=pl.BlockSpec((1000, 128), lambda i: (i, 0)),
        out_shape=jax.ShapeDtypeStruct((N_NODES, 128), jnp.float32),
    )(h0, degp, degp)


def _tc_layer(aggp, degp, h0, w):
    def body(a0_ref, a1_ref, d0_ref, d1_ref, h0_ref, w_ref, h_ref, hs_ref):
        inv = _inv_from_deg(d0_ref[0], d1_ref[0])
        support = (a0_ref[0] + a1_ref[0]) * inv + ALPHA * h0_ref[...]
        h = jnp.maximum(_matmul_t(support, w_ref[...]), 0.0)
        h_ref[...] = h
        hs_ref[...] = h * inv

    return pl.pallas_call(
        body,
        grid=(10,),
        in_specs=[
            pl.BlockSpec((1, 1000, 128), lambda i: (0, i, 0)),
            pl.BlockSpec((1, 1000, 128), lambda i: (1, i, 0)),
            pl.BlockSpec((1, 1000, 128), lambda i: (0, i, 0)),
            pl.BlockSpec((1, 1000, 128), lambda i: (1, i, 0)),
            pl.BlockSpec((1000, 128), lambda i: (i, 0)),
            pl.BlockSpec((128, 128), lambda i: (0, 0)),
        ],
        out_specs=[
            pl.BlockSpec((1000, 128), lambda i: (i, 0)),
            pl.BlockSpec((1000, 128), lambda i: (i, 0)),
        ],
        out_shape=[
            jax.ShapeDtypeStruct((N_NODES, 128), jnp.float32),
            jax.ShapeDtypeStruct((N_NODES, 128), jnp.float32),
        ],
    )(aggp, aggp, degp, degp, h0, w)


def _tc_final(aggp, degp, h0, wl, w2):
    def body(a0_ref, a1_ref, d0_ref, d1_ref, h0_ref, wl_ref, w2_ref, out_ref):
        inv = _inv_from_deg(d0_ref[0], d1_ref[0])
        support = (a0_ref[0] + a1_ref[0]) * inv + ALPHA * h0_ref[...]
        h2 = jnp.maximum(_matmul_t(support, wl_ref[...]), 0.0)
        out_ref[...] = _matmul_t(h2, w2_ref[...])

    return pl.pallas_call(
        body,
        grid=(10,),
        in_specs=[
            pl.BlockSpec((1, 1000, 128), lambda i: (0, i, 0)),
            pl.BlockSpec((1, 1000, 128), lambda i: (1, i, 0)),
            pl.BlockSpec((1, 1000, 128), lambda i: (0, i, 0)),
            pl.BlockSpec((1, 1000, 128), lambda i: (1, i, 0)),
            pl.BlockSpec((1000, 128), lambda i: (i, 0)),
            pl.BlockSpec((128, 128), lambda i: (0, 0)),
            pl.BlockSpec((64, 128), lambda i: (0, 0)),
        ],
        out_specs=pl.BlockSpec((1000, 64), lambda i: (i, 0)),
        out_shape=jax.ShapeDtypeStruct((N_NODES, 64), jnp.float32),
    )(aggp, aggp, degp, degp, h0, wl, w2)


def kernel(graph, features, W1, Wl0, Wl1, W2):
    src = graph[0].astype(jnp.int32)
    dst = graph[1].astype(jnp.int32)
    n_fill = E_PAD - N_EDGES
    fill = jnp.arange(n_fill, dtype=jnp.int32)
    src_p = jnp.concatenate([src, fill % N_NODES])
    dst_p = jnp.concatenate([dst, N_NODES + fill % (N_PAD - N_NODES)])
    src3 = src_p.reshape(NW, C, K)
    dst3 = dst_p.reshape(NW, C, K)

    ones128 = jnp.ones((K, 128), jnp.float32)
    zeros128 = jnp.zeros((N_PAD, 128), jnp.float32)

    h0 = _tc_h0(features, W1)
    degp = _deg_sc(dst3, ones128, zeros128)
    h0s = _tc_scale(h0, degp)
    aggp = _agg_sc(h0s, src3, dst3, zeros128)
    h1, h1s = _tc_layer(aggp, degp, h0, Wl0)
    aggp2 = _agg_sc(h1s, src3, dst3, zeros128)
    return _tc_final(aggp2, degp, h0, Wl1, W2)


# speedup vs baseline: 1.1002x; 1.1002x over previous
import functools

import jax
import jax.numpy as jnp
from jax import lax
from jax.experimental import pallas as pl
from jax.experimental.pallas import tpu as pltpu
from jax.experimental.pallas import tpu_sc as plsc

N_NODES = 10000
N_EDGES = 320000
D_FEAT = 128
D_HID = 128
N_CLASSES = 64
ALPHA = 1.0

NW = 32
K = 128
C = 80
E_PAD = NW * C * K
N_PAD = 10112
R = N_PAD // 16
EPW = E_PAD // NW
MB = 1264

AK = 32
SUB = K // AK
AC = EPW // AK
NBUF = 4


def _sc_mesh():
    return plsc.VectorSubcoreMesh(
        core_axis_name="c", subcore_axis_name="s", num_cores=2, num_subcores=16
    )


def _deg_sc(dst3, ones128, zeros128):
    @functools.partial(
        pl.kernel,
        out_type=jax.ShapeDtypeStruct((2, N_PAD, 128), jnp.float32),
        mesh=_sc_mesh(),
        scratch_types=[
            pltpu.VMEM_SHARED((N_PAD, 128), jnp.float32),
            pltpu.VMEM((C, K), jnp.int32),
            pltpu.VMEM((K, 128), jnp.float32),
            pltpu.SemaphoreType.DMA,
        ],
    )
    def body(dst_hbm, ones_hbm, zeros_hbm, out_hbm, dacc, dst_v, ones_v, dsem):
        cid = lax.axis_index("c")
        sid = lax.axis_index("s")
        wid = cid * 16 + sid
        pltpu.sync_copy(zeros_hbm.at[pl.ds(sid * R, R)], dacc.at[pl.ds(sid * R, R)])
        pltpu.sync_copy(dst_hbm.at[wid], dst_v)
        pltpu.sync_copy(ones_hbm, ones_v)
        plsc.subcore_barrier()

        def fire(j, carry):
            pltpu.async_copy(ones_v, dacc.at[dst_v.at[j]], dsem, add=True)
            return carry

        def drain(j, carry):
            pltpu.make_async_copy(ones_v, dacc.at[dst_v.at[j]], dsem).wait()
            return carry

        lax.fori_loop(0, C, fire, 0)
        lax.fori_loop(0, C, drain, 0)
        plsc.subcore_barrier()
        pltpu.sync_copy(
            dacc.at[pl.ds(sid * R, R)], out_hbm.at[cid, pl.ds(sid * R, R)]
        )

    return body(dst3, ones128, zeros128)


def _agg_sc(hs, src3, dst3, zeros128):
    @functools.partial(
        pl.kernel,
        out_type=jax.ShapeDtypeStruct((2, N_PAD, 128), jnp.float32),
        mesh=_sc_mesh(),
        scratch_types=[
            pltpu.VMEM_SHARED((N_PAD, 128), jnp.float32),
            pltpu.VMEM((C, K), jnp.int32),
            pltpu.VMEM((C, K), jnp.int32),
            pltpu.VMEM((NBUF, AK, 128), jnp.float32),
            [pltpu.SemaphoreType.DMA] * NBUF,
        ],
    )
    def body(hs_hbm, src_hbm, dst_hbm, zeros_hbm, out_hbm, acc, src_v, dst_v,
             rows_v, sems):
        cid = lax.axis_index("c")
        sid = lax.axis_index("s")
        wid = cid * 16 + sid
        pltpu.sync_copy(zeros_hbm.at[pl.ds(sid * R, R)], acc.at[pl.ds(sid * R, R)])
        pltpu.sync_copy(src_hbm.at[wid], src_v)
        pltpu.sync_copy(dst_hbm.at[wid], dst_v)
        plsc.subcore_barrier()

        def idx_slice(ref, j):
            return ref.at[j // SUB, pl.ds((j % SUB) * AK, AK)]

        def start(j, b):
            pltpu.async_copy(hs_hbm.at[idx_slice(src_v, j)], rows_v.at[b],
                             sems[b])

        def wait(j, b):
            pltpu.make_async_copy(hs_hbm.at[idx_slice(src_v, j)],
                                  rows_v.at[b], sems[b]).wait()

        for b in range(NBUF):
            start(b, b)

        def step(g, carry):
            for b in range(NBUF):
                j = g * NBUF + b
                wait(j, b)
                pltpu.sync_copy(rows_v.at[b], acc.at[idx_slice(dst_v, j)],
                                add=True)
                start(j + NBUF, b)
            return carry

        lax.fori_loop(0, AC // NBUF - 1, step, 0)
        for b in range(NBUF):
            j = AC - NBUF + b
            wait(j, b)
            pltpu.sync_copy(rows_v.at[b], acc.at[idx_slice(dst_v, j)],
                            add=True)

        plsc.subcore_barrier()
        pltpu.sync_copy(
            acc.at[pl.ds(sid * R, R)], out_hbm.at[cid, pl.ds(sid * R, R)]
        )

    return body(hs, src3, dst3, zeros128)


def _inv_from_deg(d0, d1):
    deg = jnp.maximum(d0[:, 0:1] + d1[:, 0:1], 1.0)
    return lax.rsqrt(deg)


def _matmul_t(x, w):
    return lax.dot_general(
        x, w, (((1,), (1,)), ((), ())), preferred_element_type=jnp.float32
    )


def _tc_h0(x, w1):
    def body(x_ref, w_ref, h0_ref):
        h0_ref[...] = jnp.maximum(_matmul_t(x_ref[...], w_ref[...]), 0.0)

    return pl.pallas_call(
        body,
        grid=(10,),
        in_specs=[
            pl.BlockSpec((1000, 128), lambda i: (i, 0)),
            pl.BlockSpec((128, 128), lambda i: (0, 0)),
        ],
        out_specs=pl.BlockSpec((1000, 128), lambda i: (i, 0)),
        out_shape=jax.ShapeDtypeStruct((N_NODES, 128), jnp.float32),
    )(x, w1)


def _tc_scale(h0, degp):
    def body(h0_ref, d0_ref, d1_ref, h0s_ref):
        inv = _inv_from_deg(d0_ref[0], d1_ref[0])
        h0s_ref[...] = h0_ref[...] * inv

    return pl.pallas_call(
        body,
        grid=(10,),
        in_specs=[
            pl.BlockSpec((1000, 128), lambda i: (i, 0)),
            pl.BlockSpec((1, 1000, 128), lambda i: (0, i, 0)),
            pl.BlockSpec((1, 1000, 128), lambda i: (1, i, 0)),
        ],
        out_specs=pl.BlockSpec((1000, 128), lambda i: (i, 0)),
        out_shape=jax.ShapeDtypeStruct((N_NODES, 128), jnp.float32),
    )(h0, degp, degp)


def _tc_layer(aggp, degp, h0, w):
    def body(a0_ref, a1_ref, d0_ref, d1_ref, h0_ref, w_ref, h_ref, hs_ref):
        inv = _inv_from_deg(d0_ref[0], d1_ref[0])
        support = (a0_ref[0] + a1_ref[0]) * inv + ALPHA * h0_ref[...]
        h = jnp.maximum(_matmul_t(support, w_ref[...]), 0.0)
        h_ref[...] = h
        hs_ref[...] = h * inv

    return pl.pallas_call(
        body,
        grid=(10,),
        in_specs=[
            pl.BlockSpec((1, 1000, 128), lambda i: (0, i, 0)),
            pl.BlockSpec((1, 1000, 128), lambda i: (1, i, 0)),
            pl.BlockSpec((1, 1000, 128), lambda i: (0, i, 0)),
            pl.BlockSpec((1, 1000, 128), lambda i: (1, i, 0)),
            pl.BlockSpec((1000, 128), lambda i: (i, 0)),
            pl.BlockSpec((128, 128), lambda i: (0, 0)),
        ],
        out_specs=[
            pl.BlockSpec((1000, 128), lambda i: (i, 0)),
            pl.BlockSpec((1000, 128), lambda i: (i, 0)),
        ],
        out_shape=[
            jax.ShapeDtypeStruct((N_NODES, 128), jnp.float32),
            jax.ShapeDtypeStruct((N_NODES, 128), jnp.float32),
        ],
    )(aggp, aggp, degp, degp, h0, w)


def _tc_final(aggp, degp, h0, wl, w2):
    def body(a0_ref, a1_ref, d0_ref, d1_ref, h0_ref, wl_ref, w2_ref, out_ref):
        inv = _inv_from_deg(d0_ref[0], d1_ref[0])
        support = (a0_ref[0] + a1_ref[0]) * inv + ALPHA * h0_ref[...]
        h2 = jnp.maximum(_matmul_t(support, wl_ref[...]), 0.0)
        out_ref[...] = _matmul_t(h2, w2_ref[...])

    return pl.pallas_call(
        body,
        grid=(10,),
        in_specs=[
            pl.BlockSpec((1, 1000, 128), lambda i: (0, i, 0)),
            pl.BlockSpec((1, 1000, 128), lambda i: (1, i, 0)),
            pl.BlockSpec((1, 1000, 128), lambda i: (0, i, 0)),
            pl.BlockSpec((1, 1000, 128), lambda i: (1, i, 0)),
            pl.BlockSpec((1000, 128), lambda i: (i, 0)),
            pl.BlockSpec((128, 128), lambda i: (0, 0)),
            pl.BlockSpec((64, 128), lambda i: (0, 0)),
        ],
        out_specs=pl.BlockSpec((1000, 64), lambda i: (i, 0)),
        out_shape=jax.ShapeDtypeStruct((N_NODES, 64), jnp.float32),
    )(aggp, aggp, degp, degp, h0, wl, w2)


def kernel(graph, features, W1, Wl0, Wl1, W2):
    src = graph[0].astype(jnp.int32)
    dst = graph[1].astype(jnp.int32)
    n_fill = E_PAD - N_EDGES
    fill = jnp.arange(n_fill, dtype=jnp.int32)
    src_p = jnp.concatenate([src, fill % N_NODES])
    dst_p = jnp.concatenate([dst, N_NODES + fill % (N_PAD - N_NODES)])
    src3 = src_p.reshape(NW, C, K)
    dst3 = dst_p.reshape(NW, C, K)

    ones128 = jnp.ones((K, 128), jnp.float32)
    zeros128 = jnp.zeros((N_PAD, 128), jnp.float32)

    h0 = _tc_h0(features, W1)
    degp = _deg_sc(dst3, ones128, zeros128)
    h0s = _tc_scale(h0, degp)
    aggp = _agg_sc(h0s, src3, dst3, zeros128)
    h1, h1s = _tc_layer(aggp, degp, h0, Wl0)
    aggp2 = _agg_sc(h1s, src3, dst3, zeros128)
    return _tc_final(aggp2, degp, h0, Wl1, W2)
